# baseline (speedup 1.0000x reference)
import functools

import jax
import jax.numpy as jnp
from jax import lax
from jax.experimental import pallas as pl
from jax.experimental.pallas import tpu as pltpu
from jax.experimental.pallas import tpu_sc as plsc

N = 2048
E = 65536
D_X = 512
D_E = 16
OUT_X = 512
OUT_E = 256

NC = 2
NS = 16
NW = NC * NS
EPW = E // NW
BCH = 128
KCH = EPW // BCH
RPT = N // NS


def _segment_sum_sc(idx, ea):
    mesh = plsc.VectorSubcoreMesh(core_axis_name="c", subcore_axis_name="s")

    @functools.partial(
        pl.kernel,
        out_type=jax.ShapeDtypeStruct((NC, N, D_E), jnp.float32),
        mesh=mesh,
        scratch_types=[
            pltpu.VMEM((KCH, BCH), jnp.int32),
            pltpu.VMEM((EPW, D_E), jnp.float32),
            pltpu.VMEM((RPT, D_E), jnp.float32),
            pltpu.VMEM_SHARED((N, D_E), jnp.float32),
        ],
        compiler_params=pltpu.CompilerParams(use_tc_tiling_on_sc=False),
    )
    def seg_kernel(idx_hbm, ea_hbm, out_hbm, idx_v, rows_v, stripe_v, acc_sh):
        c = lax.axis_index("c")
        s = lax.axis_index("s")
        wid = s * NC + c
        base = wid * EPW

        def zero_row(i, carry):
            stripe_v[i] = jnp.zeros((D_E,), jnp.float32)
            return carry

        lax.fori_loop(0, RPT, zero_row, 0)
        pltpu.sync_copy(stripe_v, acc_sh.at[pl.ds(s * RPT, RPT)])
        plsc.subcore_barrier()

        pltpu.sync_copy(ea_hbm.at[pl.ds(base, EPW)], rows_v)
        for j in range(KCH):
            pltpu.sync_copy(idx_hbm.at[0, pl.ds(base + j * BCH, BCH)],
                            idx_v.at[j])

        for j in range(KCH):
            pltpu.sync_copy(rows_v.at[pl.ds(j * BCH, BCH)],
                            acc_sh.at[idx_v.at[j]], add=True)
        plsc.subcore_barrier()

        pltpu.sync_copy(acc_sh.at[pl.ds(s * RPT, RPT)], stripe_v)
        pltpu.sync_copy(stripe_v, out_hbm.at[c, pl.ds(s * RPT, RPT)])

    return seg_kernel(idx, ea)


def _fused_linear_tc(x, Wx, bx, partials, We, be):
    def body(x_ref, wx_ref, bx_ref, p_ref, we_ref, be_ref, o_ref):
        agg = p_ref[0] + p_ref[1]
        xo = lax.dot_general(x_ref[...], wx_ref[...],
                             (((1,), (1,)), ((), ())),
                             preferred_element_type=jnp.float32)
        eo = lax.dot_general(agg, we_ref[...],
                             (((1,), (1,)), ((), ())),
                             preferred_element_type=jnp.float32)
        o_ref[:, :OUT_X] = xo + bx_ref[...][None, :]
        o_ref[:, OUT_X:] = eo + be_ref[...][None, :]

    return pl.pallas_call(
        body,
        out_shape=jax.ShapeDtypeStruct((N, OUT_X + OUT_E), jnp.float32),
    )(x, Wx, bx, partials, We, be)


def kernel(x, edge_index, edge_attr, Wx, bx, We, be):
    partials = _segment_sum_sc(edge_index.astype(jnp.int32), edge_attr)
    return _fused_linear_tc(x, Wx, bx, partials, We, be)


# speedup vs baseline: 1.7232x; 1.7232x over previous
import functools

import jax
import jax.numpy as jnp
from jax import lax
from jax.experimental import pallas as pl
from jax.experimental.pallas import tpu as pltpu
from jax.experimental.pallas import tpu_sc as plsc

N = 2048
E = 65536
D_X = 512
D_E = 16
OUT_X = 512
OUT_E = 256

NC = 2
NS = 16
NW = NC * NS
EPW = E // NW
BCH = 128
KCH = EPW // BCH
RPT = N // NS
L = 16


def _segment_sum_sc(idx3, ea_t):
    mesh = plsc.VectorSubcoreMesh(core_axis_name="c", subcore_axis_name="s")

    @functools.partial(
        pl.kernel,
        out_type=jax.ShapeDtypeStruct((NC, D_E, N), jnp.float32),
        mesh=mesh,
        scratch_types=[
            pltpu.VMEM((KCH, BCH), jnp.int32),
            pltpu.VMEM((D_E, EPW), jnp.float32),
            pltpu.VMEM((EPW, D_E), jnp.float32),
            pltpu.VMEM((RPT, D_E), jnp.float32),
            pltpu.VMEM((D_E, RPT), jnp.float32),
            pltpu.VMEM_SHARED((N, D_E), jnp.float32),
        ],
        compiler_params=pltpu.CompilerParams(use_tc_tiling_on_sc=False,
                                             needs_layout_passes=False),
    )
    def seg_kernel(idx_hbm, ea_hbm, out_hbm, idx_v, tbuf_v, rows_v,
                   stripe_v, stripet_v, acc_sh):
        c = lax.axis_index("c")
        s = lax.axis_index("s")
        wid = s * NC + c
        base = wid * EPW

        def zero_row(i, carry):
            stripe_v[i] = jnp.zeros((D_E,), jnp.float32)
            return carry

        lax.fori_loop(0, RPT, zero_row, 0)
        pltpu.sync_copy(stripe_v, acc_sh.at[pl.ds(s * RPT, RPT)])

        pltpu.sync_copy(idx_hbm.at[wid], idx_v)
        pltpu.sync_copy(ea_hbm.at[:, pl.ds(base, EPW)], tbuf_v)

        lanes = lax.iota(jnp.int32, L)

        def tr_chunk(ch, carry):
            erow = ch * L + lanes
            for d in range(D_E):
                vals = tbuf_v[d, pl.ds(ch * L, L)]
                plsc.store_scatter(rows_v, [erow, jnp.full((L,), d, jnp.int32)],
                                   vals)
            return carry

        lax.fori_loop(0, EPW // L, tr_chunk, 0, unroll=2)
        plsc.subcore_barrier()

        for j in range(KCH):
            pltpu.sync_copy(rows_v.at[pl.ds(j * BCH, BCH)],
                            acc_sh.at[idx_v.at[j]], add=True)
        plsc.subcore_barrier()

        pltpu.sync_copy(acc_sh.at[pl.ds(s * RPT, RPT)], stripe_v)

        def tr_out(ch, carry):
            rrow = ch * L + lanes
            for d in range(D_E):
                vals = plsc.load_gather(
                    stripe_v, [rrow, jnp.full((L,), d, jnp.int32)])
                stripet_v[d, pl.ds(ch * L, L)] = vals
            return carry

        lax.fori_loop(0, RPT // L, tr_out, 0, unroll=2)
        pltpu.sync_copy(stripet_v, out_hbm.at[c, :, pl.ds(s * RPT, RPT)])

    return seg_kernel(idx3, ea_t)


def _fused_linear_tc(x, Wx, bx, partials_t, We, be):
    def body(x_ref, wx_ref, bx_ref, p_ref, we_ref, be_ref, o_ref):
        agg_t = p_ref[0] + p_ref[1]
        xo = lax.dot_general(x_ref[...], wx_ref[...],
                             (((1,), (1,)), ((), ())),
                             preferred_element_type=jnp.float32)
        eo = lax.dot_general(agg_t, we_ref[...],
                             (((0,), (1,)), ((), ())),
                             preferred_element_type=jnp.float32)
        o_ref[:, :OUT_X] = xo + bx_ref[...][None, :]
        o_ref[:, OUT_X:] = eo + be_ref[...][None, :]

    return pl.pallas_call(
        body,
        out_shape=jax.ShapeDtypeStruct((N, OUT_X + OUT_E), jnp.float32),
    )(x, Wx, bx, partials_t, We, be)


def kernel(x, edge_index, edge_attr, Wx, bx, We, be):
    idx3 = edge_index[0].astype(jnp.int32).reshape(NW, KCH, BCH)
    ea_t = edge_attr.T
    partials_t = _segment_sum_sc(idx3, ea_t)
    return _fused_linear_tc(x, Wx, bx, partials_t, We, be)
